# baseline (speedup 1.0000x reference)
import functools
import math

import jax
import jax.numpy as jnp
from jax import lax
from jax.experimental import pallas as pl
from jax.experimental.pallas import tpu as pltpu
from jax.experimental.pallas import tpu_sc as plsc

N = 10000
E = 320000
IN_CH = 128
HID = 128
OUT = 64
ED = 16
BN_EPS = 1e-5
L2_EPS = 1e-12

NC = 2
NS = 16
NW = NC * NS
EPT = E // NW
CH = 40
NCHUNK = EPT // CH
NBUF = 8
NGRP = NCHUNK // NBUF
NEPI = NCHUNK - NGRP * NBUF
CHS = 80
NCHUNK_S = EPT // CHS
NBUF_S = 4
NGRP_S = NCHUNK_S // NBUF_S
NEPI_S = NCHUNK_S - NGRP_S * NBUF_S
RB = 624
ZR = 26
NZ = RB // ZR
TAIL = N - NS * RB
TBASE = NS * RB

ECNB = 4
NGRP_E = NCHUNK // ECNB
NEPI_E = NCHUNK - NGRP_E * ECNB


def _zero_zb(zb, nrow):
    def row(r, carry):
        for cb in range(8):
            zb[r, pl.ds(cb * 16, 16)] = jnp.zeros((16,), jnp.float32)
        return carry
    lax.fori_loop(0, nrow, row, 0)


def _zero_shared_slice(zb, sh, rbase, s):
    for k in range(NZ):
        pltpu.sync_copy(zb, sh.at[pl.ds(rbase + k * ZR, ZR)])

    @pl.when(s == NS - 1)
    def _():
        pltpu.sync_copy(zb.at[pl.ds(0, TAIL)], sh.at[pl.ds(TBASE, TAIL)])


def _write_out_slice(sh, out, c, rbase, s):
    pltpu.sync_copy(sh.at[pl.ds(rbase, RB)], out.at[c, pl.ds(rbase, RB)])

    @pl.when(s == NS - 1)
    def _():
        pltpu.sync_copy(sh.at[pl.ds(TBASE, TAIL)],
                        out.at[c, pl.ds(TBASE, TAIL)])


def _sc_scatter_body(tab_hbm, src_hbm, dst_hbm, s_out,
                     sidx, didx, rows, zb, s_sh, semg, sems):
    c = lax.axis_index("c")
    s = lax.axis_index("s")
    wid = s * NC + c
    ebase = wid * EPT
    rbase = s * RB

    _zero_zb(zb, ZR)
    _zero_shared_slice(zb, s_sh, rbase, s)
    plsc.subcore_barrier()

    def group(g, carry):
        base0 = ebase + g * (NBUF_S * CHS)
        for p in range(NBUF_S):
            b = base0 + p * CHS

            @pl.when(g > 0)
            def _(p=p):
                pltpu.make_async_copy(
                    rows.at[p], s_sh.at[didx.at[p]], sems.at[p]).wait()
            pltpu.async_copy(src_hbm.at[pl.ds(b, CHS)], sidx.at[p],
                             semg.at[p])
            pltpu.async_copy(dst_hbm.at[pl.ds(b, CHS)], didx.at[p],
                             semg.at[p])
        for p in range(NBUF_S):
            b = base0 + p * CHS
            pltpu.make_async_copy(
                src_hbm.at[pl.ds(b, CHS)], sidx.at[p], semg.at[p]).wait()
            pltpu.make_async_copy(
                dst_hbm.at[pl.ds(b, CHS)], didx.at[p], semg.at[p]).wait()
            pltpu.async_copy(tab_hbm.at[sidx.at[p]], rows.at[p], semg.at[p])
        for p in range(NBUF_S):
            pltpu.make_async_copy(
                tab_hbm.at[sidx.at[p]], rows.at[p], semg.at[p]).wait()
            pltpu.async_copy(rows.at[p], s_sh.at[didx.at[p]], sems.at[p],
                             add=True)
        return carry
    lax.fori_loop(0, NGRP_S, group, 0)

    for p in range(NBUF_S):
        pltpu.make_async_copy(
            rows.at[p], s_sh.at[didx.at[p]], sems.at[p]).wait()

    base = ebase + NGRP_S * NBUF_S * CHS
    for q in range(NEPI_S):
        b = base + q * CHS
        pltpu.sync_copy(src_hbm.at[pl.ds(b, CHS)], sidx.at[0])
        pltpu.sync_copy(dst_hbm.at[pl.ds(b, CHS)], didx.at[0])
        pltpu.async_copy(tab_hbm.at[sidx.at[0]], rows.at[0],
                         semg.at[0]).wait()
        pltpu.sync_copy(rows.at[0], s_sh.at[didx.at[0]], add=True)

    plsc.subcore_barrier()
    _write_out_slice(s_sh, s_out, c, rbase, s)


def _sc_ec_body(dst_hbm, ea_hbm, ec_out,
                didx, eab, ecb, zb, ec_sh, semg, sems):
    c = lax.axis_index("c")
    s = lax.axis_index("s")
    wid = s * NC + c
    ebase = wid * EPT
    rbase = s * RB

    _zero_zb(zb, ZR)
    _zero_shared_slice(zb, ec_sh, rbase, s)

    one16 = jnp.where(lax.iota(jnp.int32, 16) < 1, 1.0, 0.0).astype(jnp.float32)
    z16 = jnp.zeros((16,), jnp.float32)

    def init_row(r, carry):
        for p in range(ECNB):
            ecb[p, r, pl.ds(0, 16)] = z16
            ecb[p, r, pl.ds(16, 16)] = one16
            for cb in range(2, 8):
                ecb[p, r, pl.ds(cb * 16, 16)] = z16
        return carry
    lax.fori_loop(0, CH, init_row, 0)
    plsc.subcore_barrier()

    def fill(p):
        def frow(r, carry):
            ecb[p, r, pl.ds(0, 16)] = eab[p, r]
            return carry
        lax.fori_loop(0, CH, frow, 0)

    def group(g, carry):
        base0 = ebase + g * (ECNB * CH)
        for p in range(ECNB):
            b = base0 + p * CH

            @pl.when(g > 0)
            def _(p=p):
                pltpu.make_async_copy(
                    ecb.at[p], ec_sh.at[didx.at[p]], sems.at[p]).wait()
            pltpu.async_copy(dst_hbm.at[pl.ds(b, CH)], didx.at[p],
                             semg.at[p])
            pltpu.async_copy(ea_hbm.at[pl.ds(b, CH)], eab.at[p],
                             semg.at[p])
        for p in range(ECNB):
            b = base0 + p * CH
            pltpu.make_async_copy(
                dst_hbm.at[pl.ds(b, CH)], didx.at[p], semg.at[p]).wait()
            pltpu.make_async_copy(
                ea_hbm.at[pl.ds(b, CH)], eab.at[p], semg.at[p]).wait()
            fill(p)
            pltpu.async_copy(ecb.at[p], ec_sh.at[didx.at[p]], sems.at[p],
                             add=True)
        return carry
    lax.fori_loop(0, NGRP_E, group, 0)

    for p in range(ECNB):
        pltpu.make_async_copy(
            ecb.at[p], ec_sh.at[didx.at[p]], sems.at[p]).wait()

    base = ebase + NGRP_E * ECNB * CH
    for q in range(NEPI_E):
        b = base + q * CH
        pltpu.sync_copy(dst_hbm.at[pl.ds(b, CH)], didx.at[0])
        pltpu.sync_copy(ea_hbm.at[pl.ds(b, CH)], eab.at[0])
        fill(0)
        pltpu.sync_copy(ecb.at[0], ec_sh.at[didx.at[0]], add=True)

    plsc.subcore_barrier()
    _write_out_slice(ec_sh, ec_out, c, rbase, s)


@functools.cache
def _build_sc_kernels():
    mesh = plsc.VectorSubcoreMesh(
        core_axis_name="c", subcore_axis_name="s",
        num_cores=NC, num_subcores=NS)
    scatter = pl.kernel(
        _sc_scatter_body,
        out_type=[jax.ShapeDtypeStruct((NC, N, HID), jnp.float32)],
        mesh=mesh,
        scratch_types=[
            pltpu.VMEM((NBUF_S, CHS), jnp.int32),
            pltpu.VMEM((NBUF_S, CHS), jnp.int32),
            pltpu.VMEM((NBUF_S, CHS, HID), jnp.float32),
            pltpu.VMEM((ZR, HID), jnp.float32),
            pltpu.VMEM_SHARED((N, HID), jnp.float32),
            pltpu.SemaphoreType.DMA((NBUF_S,)),
            pltpu.SemaphoreType.DMA((NBUF_S,)),
        ],
    )
    ec = pl.kernel(
        _sc_ec_body,
        out_type=[jax.ShapeDtypeStruct((NC, N, HID), jnp.float32)],
        mesh=mesh,
        scratch_types=[
            pltpu.VMEM((ECNB, CH), jnp.int32),
            pltpu.VMEM((ECNB, CH, ED), jnp.float32),
            pltpu.VMEM((ECNB, CH, HID), jnp.float32),
            pltpu.VMEM((ZR, HID), jnp.float32),
            pltpu.VMEM_SHARED((N, HID), jnp.float32),
            pltpu.SemaphoreType.DMA((ECNB,)),
            pltpu.SemaphoreType.DMA((ECNB,)),
        ],
    )
    return scatter, ec


BLK = 400
_BN_SCALE = 1.0 / math.sqrt(1.0 + BN_EPS)


def _tc1_body(s_ref, e_ref, x_ref, wn_ref, we_ref, ws_ref,
              b_ref, g_ref, bt_ref, o_ref):
    S = s_ref[0] + s_ref[1]
    EC = e_ref[0] + e_ref[1]
    Es = EC[:, 0:ED]
    cnt = EC[:, ED:ED + 1]
    agg = (jnp.dot(S, wn_ref[...], preferred_element_type=jnp.float32)
           + jnp.dot(Es, we_ref[...], preferred_element_type=jnp.float32))
    agg = agg / jnp.maximum(cnt, 1.0)
    h = agg + jnp.dot(x_ref[...], ws_ref[...],
                      preferred_element_type=jnp.float32) + b_ref[...]
    h = h * (g_ref[...] * _BN_SCALE) + bt_ref[...]
    h = jnp.maximum(h, 0.0)
    nrm = jnp.sqrt(jnp.sum(h * h, axis=1, keepdims=True))
    o_ref[...] = h / jnp.maximum(nrm, L2_EPS)


def _tc2_body(s_ref, e_ref, h_ref, wn_ref, we_ref, ws_ref,
              b_ref, o_ref):
    S = s_ref[0] + s_ref[1]
    EC = e_ref[0] + e_ref[1]
    Es = EC[:, 0:ED]
    cnt = EC[:, ED:ED + 1]
    agg = (jnp.dot(S, wn_ref[...], preferred_element_type=jnp.float32)
           + jnp.dot(Es, we_ref[...], preferred_element_type=jnp.float32))
    agg = agg / jnp.maximum(cnt, 1.0)
    z = agg + jnp.dot(h_ref[...], ws_ref[...],
                      preferred_element_type=jnp.float32) + b_ref[...]
    z = jnp.maximum(z, 0.0)
    nrm = jnp.sqrt(jnp.sum(z * z, axis=1, keepdims=True))
    o_ref[...] = z / jnp.maximum(nrm, L2_EPS)


def _full(shape):
    nd = len(shape)
    return pl.BlockSpec(shape, lambda i: (0,) * nd)


_tc1 = pl.pallas_call(
    _tc1_body,
    grid=(N // BLK,),
    in_specs=[
        pl.BlockSpec((NC, BLK, HID), lambda i: (0, i, 0)),
        pl.BlockSpec((NC, BLK, HID), lambda i: (0, i, 0)),
        pl.BlockSpec((BLK, IN_CH), lambda i: (i, 0)),
        _full((IN_CH, HID)),
        _full((ED, HID)),
        _full((IN_CH, HID)),
        _full((1, HID)),
        _full((1, HID)),
        _full((1, HID)),
    ],
    out_specs=pl.BlockSpec((BLK, HID), lambda i: (i, 0)),
    out_shape=jax.ShapeDtypeStruct((N, HID), jnp.float32),
)

_tc2 = pl.pallas_call(
    _tc2_body,
    grid=(N // BLK,),
    in_specs=[
        pl.BlockSpec((NC, BLK, HID), lambda i: (0, i, 0)),
        pl.BlockSpec((NC, BLK, HID), lambda i: (0, i, 0)),
        pl.BlockSpec((BLK, HID), lambda i: (i, 0)),
        _full((HID, OUT)),
        _full((ED, OUT)),
        _full((HID, OUT)),
        _full((1, OUT)),
    ],
    out_specs=pl.BlockSpec((BLK, OUT), lambda i: (i, 0)),
    out_shape=jax.ShapeDtypeStruct((N, OUT), jnp.float32),
)


def kernel(x, edge_index, edge_attr, Wn1, We1, Ws1, b1, gamma, beta,
           Wn2, We2, Ws2, b2):
    _sc_scatter, _sc_ec = _build_sc_kernels()
    src = edge_index[0].astype(jnp.int32)
    dst = edge_index[1].astype(jnp.int32)
    (EC,) = _sc_ec(dst, edge_attr)
    (S1,) = _sc_scatter(x, src, dst)
    h = _tc1(S1, EC, x, Wn1, We1, Ws1,
             b1.reshape(1, HID), gamma.reshape(1, HID), beta.reshape(1, HID))
    (S2,) = _sc_scatter(h, src, dst)
    z = _tc2(S2, EC, h, Wn2, We2, Ws2, b2.reshape(1, OUT))
    return z


# speedup vs baseline: 1.0075x; 1.0075x over previous
import functools
import math

import jax
import jax.numpy as jnp
from jax import lax
from jax.experimental import pallas as pl
from jax.experimental.pallas import tpu as pltpu
from jax.experimental.pallas import tpu_sc as plsc

N = 10000
E = 320000
IN_CH = 128
HID = 128
OUT = 64
ED = 16
BN_EPS = 1e-5
L2_EPS = 1e-12

NC = 2
NS = 16
NW = NC * NS
EPT = E // NW
CH = 40
NCHUNK = EPT // CH
NBUF = 8
NGRP = NCHUNK // NBUF
NEPI = NCHUNK - NGRP * NBUF
CHS = 40
NCHUNK_S = EPT // CHS
NBUF_S = 8
NGRP_S = NCHUNK_S // NBUF_S
NEPI_S = NCHUNK_S - NGRP_S * NBUF_S
RB = 624
ZR = 26
NZ = RB // ZR
TAIL = N - NS * RB
TBASE = NS * RB

ECNB = 4
NGRP_E = NCHUNK // ECNB
NEPI_E = NCHUNK - NGRP_E * ECNB


def _zero_zb(zb, nrow):
    def row(r, carry):
        for cb in range(8):
            zb[r, pl.ds(cb * 16, 16)] = jnp.zeros((16,), jnp.float32)
        return carry
    lax.fori_loop(0, nrow, row, 0)


def _zero_shared_slice(zb, sh, rbase, s):
    for k in range(NZ):
        pltpu.sync_copy(zb, sh.at[pl.ds(rbase + k * ZR, ZR)])

    @pl.when(s == NS - 1)
    def _():
        pltpu.sync_copy(zb.at[pl.ds(0, TAIL)], sh.at[pl.ds(TBASE, TAIL)])


def _write_out_slice(sh, out, c, rbase, s):
    pltpu.sync_copy(sh.at[pl.ds(rbase, RB)], out.at[c, pl.ds(rbase, RB)])

    @pl.when(s == NS - 1)
    def _():
        pltpu.sync_copy(sh.at[pl.ds(TBASE, TAIL)],
                        out.at[c, pl.ds(TBASE, TAIL)])


def _sc_scatter_body(tab_hbm, src_hbm, dst_hbm, s_out,
                     sidx, didx, rows, zb, s_sh, semg, sems):
    c = lax.axis_index("c")
    s = lax.axis_index("s")
    wid = s * NC + c
    ebase = wid * EPT
    rbase = s * RB

    _zero_zb(zb, ZR)
    _zero_shared_slice(zb, s_sh, rbase, s)
    plsc.subcore_barrier()

    def group(g, carry):
        base0 = ebase + g * (NBUF_S * CHS)
        for p in range(NBUF_S):
            b = base0 + p * CHS

            @pl.when(g > 0)
            def _(p=p):
                pltpu.make_async_copy(
                    rows.at[p], s_sh.at[didx.at[p]], sems.at[p]).wait()
            pltpu.async_copy(src_hbm.at[pl.ds(b, CHS)], sidx.at[p],
                             semg.at[p])
            pltpu.async_copy(dst_hbm.at[pl.ds(b, CHS)], didx.at[p],
                             semg.at[p])
        for p in range(NBUF_S):
            b = base0 + p * CHS
            pltpu.make_async_copy(
                src_hbm.at[pl.ds(b, CHS)], sidx.at[p], semg.at[p]).wait()
            pltpu.make_async_copy(
                dst_hbm.at[pl.ds(b, CHS)], didx.at[p], semg.at[p]).wait()
            pltpu.async_copy(tab_hbm.at[sidx.at[p]], rows.at[p], semg.at[p])
        for p in range(NBUF_S):
            pltpu.make_async_copy(
                tab_hbm.at[sidx.at[p]], rows.at[p], semg.at[p]).wait()
            pltpu.async_copy(rows.at[p], s_sh.at[didx.at[p]], sems.at[p],
                             add=True)
        return carry
    lax.fori_loop(0, NGRP_S, group, 0)

    for p in range(NBUF_S):
        pltpu.make_async_copy(
            rows.at[p], s_sh.at[didx.at[p]], sems.at[p]).wait()

    base = ebase + NGRP_S * NBUF_S * CHS
    for q in range(NEPI_S):
        b = base + q * CHS
        pltpu.sync_copy(src_hbm.at[pl.ds(b, CHS)], sidx.at[0])
        pltpu.sync_copy(dst_hbm.at[pl.ds(b, CHS)], didx.at[0])
        pltpu.async_copy(tab_hbm.at[sidx.at[0]], rows.at[0],
                         semg.at[0]).wait()
        pltpu.sync_copy(rows.at[0], s_sh.at[didx.at[0]], add=True)

    plsc.subcore_barrier()
    _write_out_slice(s_sh, s_out, c, rbase, s)


def _sc_ec_body(dst_hbm, ea_hbm, ec_out,
                didx, eab, ecb, zb, ec_sh, semg, sems):
    c = lax.axis_index("c")
    s = lax.axis_index("s")
    wid = s * NC + c
    ebase = wid * EPT
    rbase = s * RB

    _zero_zb(zb, ZR)
    _zero_shared_slice(zb, ec_sh, rbase, s)

    one16 = jnp.where(lax.iota(jnp.int32, 16) < 1, 1.0, 0.0).astype(jnp.float32)
    z16 = jnp.zeros((16,), jnp.float32)

    def init_row(r, carry):
        for p in range(ECNB):
            ecb[p, r, pl.ds(0, 16)] = z16
            ecb[p, r, pl.ds(16, 16)] = one16
            for cb in range(2, 8):
                ecb[p, r, pl.ds(cb * 16, 16)] = z16
        return carry
    lax.fori_loop(0, CH, init_row, 0)
    plsc.subcore_barrier()

    def fill(p):
        def frow(r, carry):
            ecb[p, r, pl.ds(0, 16)] = eab[p, r]
            return carry
        lax.fori_loop(0, CH, frow, 0)

    def group(g, carry):
        base0 = ebase + g * (ECNB * CH)
        for p in range(ECNB):
            b = base0 + p * CH

            @pl.when(g > 0)
            def _(p=p):
                pltpu.make_async_copy(
                    ecb.at[p], ec_sh.at[didx.at[p]], sems.at[p]).wait()
            pltpu.async_copy(dst_hbm.at[pl.ds(b, CH)], didx.at[p],
                             semg.at[p])
            pltpu.async_copy(ea_hbm.at[pl.ds(b, CH)], eab.at[p],
                             semg.at[p])
        for p in range(ECNB):
            b = base0 + p * CH
            pltpu.make_async_copy(
                dst_hbm.at[pl.ds(b, CH)], didx.at[p], semg.at[p]).wait()
            pltpu.make_async_copy(
                ea_hbm.at[pl.ds(b, CH)], eab.at[p], semg.at[p]).wait()
            fill(p)
            pltpu.async_copy(ecb.at[p], ec_sh.at[didx.at[p]], sems.at[p],
                             add=True)
        return carry
    lax.fori_loop(0, NGRP_E, group, 0)

    for p in range(ECNB):
        pltpu.make_async_copy(
            ecb.at[p], ec_sh.at[didx.at[p]], sems.at[p]).wait()

    base = ebase + NGRP_E * ECNB * CH
    for q in range(NEPI_E):
        b = base + q * CH
        pltpu.sync_copy(dst_hbm.at[pl.ds(b, CH)], didx.at[0])
        pltpu.sync_copy(ea_hbm.at[pl.ds(b, CH)], eab.at[0])
        fill(0)
        pltpu.sync_copy(ecb.at[0], ec_sh.at[didx.at[0]], add=True)

    plsc.subcore_barrier()
    _write_out_slice(ec_sh, ec_out, c, rbase, s)


@functools.cache
def _build_sc_kernels():
    mesh = plsc.VectorSubcoreMesh(
        core_axis_name="c", subcore_axis_name="s",
        num_cores=NC, num_subcores=NS)
    scatter = pl.kernel(
        _sc_scatter_body,
        out_type=[jax.ShapeDtypeStruct((NC, N, HID), jnp.float32)],
        mesh=mesh,
        scratch_types=[
            pltpu.VMEM((NBUF_S, CHS), jnp.int32),
            pltpu.VMEM((NBUF_S, CHS), jnp.int32),
            pltpu.VMEM((NBUF_S, CHS, HID), jnp.float32),
            pltpu.VMEM((ZR, HID), jnp.float32),
            pltpu.VMEM_SHARED((N, HID), jnp.float32),
            pltpu.SemaphoreType.DMA((NBUF_S,)),
            pltpu.SemaphoreType.DMA((NBUF_S,)),
        ],
    )
    ec = pl.kernel(
        _sc_ec_body,
        out_type=[jax.ShapeDtypeStruct((NC, N, HID), jnp.float32)],
        mesh=mesh,
        scratch_types=[
            pltpu.VMEM((ECNB, CH), jnp.int32),
            pltpu.VMEM((ECNB, CH, ED), jnp.float32),
            pltpu.VMEM((ECNB, CH, HID), jnp.float32),
            pltpu.VMEM((ZR, HID), jnp.float32),
            pltpu.VMEM_SHARED((N, HID), jnp.float32),
            pltpu.SemaphoreType.DMA((ECNB,)),
            pltpu.SemaphoreType.DMA((ECNB,)),
        ],
    )
    return scatter, ec


BLK = 400
_BN_SCALE = 1.0 / math.sqrt(1.0 + BN_EPS)


def _tc1_body(s_ref, e_ref, x_ref, wn_ref, we_ref, ws_ref,
              b_ref, g_ref, bt_ref, o_ref):
    S = s_ref[0] + s_ref[1]
    EC = e_ref[0] + e_ref[1]
    Es = EC[:, 0:ED]
    cnt = EC[:, ED:ED + 1]
    agg = (jnp.dot(S, wn_ref[...], preferred_element_type=jnp.float32)
           + jnp.dot(Es, we_ref[...], preferred_element_type=jnp.float32))
    agg = agg / jnp.maximum(cnt, 1.0)
    h = agg + jnp.dot(x_ref[...], ws_ref[...],
                      preferred_element_type=jnp.float32) + b_ref[...]
    h = h * (g_ref[...] * _BN_SCALE) + bt_ref[...]
    h = jnp.maximum(h, 0.0)
    nrm = jnp.sqrt(jnp.sum(h * h, axis=1, keepdims=True))
    o_ref[...] = h / jnp.maximum(nrm, L2_EPS)


def _tc2_body(s_ref, e_ref, h_ref, wn_ref, we_ref, ws_ref,
              b_ref, o_ref):
    S = s_ref[0] + s_ref[1]
    EC = e_ref[0] + e_ref[1]
    Es = EC[:, 0:ED]
    cnt = EC[:, ED:ED + 1]
    agg = (jnp.dot(S, wn_ref[...], preferred_element_type=jnp.float32)
           + jnp.dot(Es, we_ref[...], preferred_element_type=jnp.float32))
    agg = agg / jnp.maximum(cnt, 1.0)
    z = agg + jnp.dot(h_ref[...], ws_ref[...],
                      preferred_element_type=jnp.float32) + b_ref[...]
    z = jnp.maximum(z, 0.0)
    nrm = jnp.sqrt(jnp.sum(z * z, axis=1, keepdims=True))
    o_ref[...] = z / jnp.maximum(nrm, L2_EPS)


def _full(shape):
    nd = len(shape)
    return pl.BlockSpec(shape, lambda i: (0,) * nd)


_tc1 = pl.pallas_call(
    _tc1_body,
    grid=(N // BLK,),
    in_specs=[
        pl.BlockSpec((NC, BLK, HID), lambda i: (0, i, 0)),
        pl.BlockSpec((NC, BLK, HID), lambda i: (0, i, 0)),
        pl.BlockSpec((BLK, IN_CH), lambda i: (i, 0)),
        _full((IN_CH, HID)),
        _full((ED, HID)),
        _full((IN_CH, HID)),
        _full((1, HID)),
        _full((1, HID)),
        _full((1, HID)),
    ],
    out_specs=pl.BlockSpec((BLK, HID), lambda i: (i, 0)),
    out_shape=jax.ShapeDtypeStruct((N, HID), jnp.float32),
)

_tc2 = pl.pallas_call(
    _tc2_body,
    grid=(N // BLK,),
    in_specs=[
        pl.BlockSpec((NC, BLK, HID), lambda i: (0, i, 0)),
        pl.BlockSpec((NC, BLK, HID), lambda i: (0, i, 0)),
        pl.BlockSpec((BLK, HID), lambda i: (i, 0)),
        _full((HID, OUT)),
        _full((ED, OUT)),
        _full((HID, OUT)),
        _full((1, OUT)),
    ],
    out_specs=pl.BlockSpec((BLK, OUT), lambda i: (i, 0)),
    out_shape=jax.ShapeDtypeStruct((N, OUT), jnp.float32),
)


def kernel(x, edge_index, edge_attr, Wn1, We1, Ws1, b1, gamma, beta,
           Wn2, We2, Ws2, b2):
    _sc_scatter, _sc_ec = _build_sc_kernels()
    src = edge_index[0].astype(jnp.int32)
    dst = edge_index[1].astype(jnp.int32)
    (EC,) = _sc_ec(dst, edge_attr)
    (S1,) = _sc_scatter(x, src, dst)
    h = _tc1(S1, EC, x, Wn1, We1, Ws1,
             b1.reshape(1, HID), gamma.reshape(1, HID), beta.reshape(1, HID))
    (S2,) = _sc_scatter(h, src, dst)
    z = _tc2(S2, EC, h, Wn2, We2, Ws2, b2.reshape(1, OUT))
    return z
